# Initial kernel scaffold; baseline (speedup 1.0000x reference)
import jax
import jax.numpy as jnp
from jax.experimental import pallas as pl


def kernel(x, adj, W1, b1, W2, b2, W3, b3, W4, b4, W5, b5, W6, b6):
    raise NotImplementedError("write your pallas kernel here")


# speedup vs baseline: 1.1523x; 1.1523x over previous
import jax
import jax.numpy as jnp
from jax.experimental import pallas as pl

N = 10000
BI = 400


def _support_body(h_ref, w_ref, o_ref):
    o_ref[...] = jnp.dot(h_ref[...], w_ref[...],
                         preferred_element_type=jnp.float32).astype(jnp.bfloat16)


def _layer_body(adj_ref, s_ref, b_ref, w_ref, o_ref):
    acc = jnp.dot(adj_ref[...], s_ref[...], preferred_element_type=jnp.float32)
    h = jnp.maximum(acc + b_ref[...], 0.0)
    o_ref[...] = jnp.dot(h, w_ref[...],
                         preferred_element_type=jnp.float32).astype(jnp.bfloat16)


def _final_body(adj_ref, s_ref, b_ref, o_ref):
    acc = jnp.dot(adj_ref[...], s_ref[...], preferred_element_type=jnp.float32)
    o_ref[...] = acc + b_ref[...]


def _support(h, w):
    fo = w.shape[1]
    return pl.pallas_call(
        _support_body,
        out_shape=jax.ShapeDtypeStruct((N, fo), jnp.bfloat16),
    )(h, w)


def _layer(adj_bf, s, b, w_next):
    fi = s.shape[1]
    fo = w_next.shape[1]
    return pl.pallas_call(
        _layer_body,
        grid=(N // BI,),
        in_specs=[
            pl.BlockSpec((BI, N), lambda i: (i, 0)),
            pl.BlockSpec((N, fi), lambda i: (0, 0)),
            pl.BlockSpec((1, fi), lambda i: (0, 0)),
            pl.BlockSpec((fi, fo), lambda i: (0, 0)),
        ],
        out_specs=pl.BlockSpec((BI, fo), lambda i: (i, 0)),
        out_shape=jax.ShapeDtypeStruct((N, fo), jnp.bfloat16),
    )(adj_bf, s, b.reshape(1, fi), w_next)


def _final(adj_bf, s, b):
    fo = s.shape[1]
    return pl.pallas_call(
        _final_body,
        grid=(N // BI,),
        in_specs=[
            pl.BlockSpec((BI, N), lambda i: (i, 0)),
            pl.BlockSpec((N, fo), lambda i: (0, 0)),
            pl.BlockSpec((1, fo), lambda i: (0, 0)),
        ],
        out_specs=pl.BlockSpec((BI, fo), lambda i: (i, 0)),
        out_shape=jax.ShapeDtypeStruct((N, fo), jnp.float32),
    )(adj_bf, s, b.reshape(1, fo))


def kernel(x, adj, W1, b1, W2, b2, W3, b3, W4, b4, W5, b5, W6, b6):
    adj_bf = adj.astype(jnp.bfloat16)
    s = _support(x, W1)
    s = _layer(adj_bf, s, b1, W2)
    s = _layer(adj_bf, s, b2, W3)
    s = _layer(adj_bf, s, b3, W4)
    s = _layer(adj_bf, s, b4, W5)
    s = _layer(adj_bf, s, b5, W6)
    return _final(adj_bf, s, b6)


# speedup vs baseline: 1.3014x; 1.1294x over previous
import jax
import jax.numpy as jnp
from jax.experimental import pallas as pl

N = 10000
BI = 400


def _support_body(h_ref, w_ref, o_ref):
    o_ref[...] = jnp.dot(h_ref[...], w_ref[...],
                         preferred_element_type=jnp.float32).astype(jnp.bfloat16)


def _layer1_body(adj_ref, s_ref, b_ref, w_ref, o_ref, adjbf_ref):
    a = adj_ref[...].astype(jnp.bfloat16)
    adjbf_ref[...] = a
    acc = jnp.dot(a, s_ref[...], preferred_element_type=jnp.float32)
    h = jnp.maximum(acc + b_ref[...], 0.0)
    o_ref[...] = jnp.dot(h, w_ref[...],
                         preferred_element_type=jnp.float32).astype(jnp.bfloat16)


def _layer_body(adj_ref, s_ref, b_ref, w_ref, o_ref):
    acc = jnp.dot(adj_ref[...], s_ref[...], preferred_element_type=jnp.float32)
    h = jnp.maximum(acc + b_ref[...], 0.0)
    o_ref[...] = jnp.dot(h, w_ref[...],
                         preferred_element_type=jnp.float32).astype(jnp.bfloat16)


def _final_body(adj_ref, s_ref, b_ref, o_ref):
    acc = jnp.dot(adj_ref[...], s_ref[...], preferred_element_type=jnp.float32)
    o_ref[...] = acc + b_ref[...]


def _support(h, w):
    fo = w.shape[1]
    return pl.pallas_call(
        _support_body,
        out_shape=jax.ShapeDtypeStruct((N, fo), jnp.bfloat16),
    )(h, w)


def _layer1(adj, s, b, w_next):
    fi = s.shape[1]
    fo = w_next.shape[1]
    return pl.pallas_call(
        _layer1_body,
        grid=(N // BI,),
        in_specs=[
            pl.BlockSpec((BI, N), lambda i: (i, 0)),
            pl.BlockSpec((N, fi), lambda i: (0, 0)),
            pl.BlockSpec((1, fi), lambda i: (0, 0)),
            pl.BlockSpec((fi, fo), lambda i: (0, 0)),
        ],
        out_specs=[
            pl.BlockSpec((BI, fo), lambda i: (i, 0)),
            pl.BlockSpec((BI, N), lambda i: (i, 0)),
        ],
        out_shape=[
            jax.ShapeDtypeStruct((N, fo), jnp.bfloat16),
            jax.ShapeDtypeStruct((N, N), jnp.bfloat16),
        ],
    )(adj, s, b.reshape(1, fi), w_next)


def _layer(adj_bf, s, b, w_next):
    fi = s.shape[1]
    fo = w_next.shape[1]
    return pl.pallas_call(
        _layer_body,
        grid=(N // BI,),
        in_specs=[
            pl.BlockSpec((BI, N), lambda i: (i, 0)),
            pl.BlockSpec((N, fi), lambda i: (0, 0)),
            pl.BlockSpec((1, fi), lambda i: (0, 0)),
            pl.BlockSpec((fi, fo), lambda i: (0, 0)),
        ],
        out_specs=pl.BlockSpec((BI, fo), lambda i: (i, 0)),
        out_shape=jax.ShapeDtypeStruct((N, fo), jnp.bfloat16),
    )(adj_bf, s, b.reshape(1, fi), w_next)


def _final(adj_bf, s, b):
    fo = s.shape[1]
    return pl.pallas_call(
        _final_body,
        grid=(N // BI,),
        in_specs=[
            pl.BlockSpec((BI, N), lambda i: (i, 0)),
            pl.BlockSpec((N, fo), lambda i: (0, 0)),
            pl.BlockSpec((1, fo), lambda i: (0, 0)),
        ],
        out_specs=pl.BlockSpec((BI, fo), lambda i: (i, 0)),
        out_shape=jax.ShapeDtypeStruct((N, fo), jnp.float32),
    )(adj_bf, s, b.reshape(1, fo))


def kernel(x, adj, W1, b1, W2, b2, W3, b3, W4, b4, W5, b5, W6, b6):
    s = _support(x, W1)
    s, adj_bf = _layer1(adj, s, b1, W2)
    s = _layer(adj_bf, s, b2, W3)
    s = _layer(adj_bf, s, b3, W4)
    s = _layer(adj_bf, s, b4, W5)
    s = _layer(adj_bf, s, b5, W6)
    return _final(adj_bf, s, b6)


# speedup vs baseline: 1.4548x; 1.1178x over previous
import jax
import jax.numpy as jnp
from jax.experimental import pallas as pl

N = 10000
BI = 400


def _support_body(h_ref, w_ref, o_ref):
    o_ref[...] = jnp.dot(h_ref[...], w_ref[...],
                         preferred_element_type=jnp.float32).astype(jnp.bfloat16)


def _stats_accumulate(sn, cs_ref, am_ref):
    cs = jnp.sum(sn, axis=0, keepdims=True)
    am = jnp.max(jnp.abs(sn), axis=0, keepdims=True)
    i = pl.program_id(0)

    @pl.when(i == 0)
    def _():
        cs_ref[...] = cs
        am_ref[...] = am

    @pl.when(i > 0)
    def _():
        cs_ref[...] = cs_ref[...] + cs
        am_ref[...] = jnp.maximum(am_ref[...], am)


def _layer1_body(adj_ref, s_ref, b_ref, w_ref,
                 snext_ref, cs_ref, am_ref, qadj_ref):
    a32 = adj_ref[...]
    qadj_ref[...] = jnp.clip(jnp.round(a32 * 254.0 - 127.0),
                             -127.0, 127.0).astype(jnp.int8)
    acc = jnp.dot(a32.astype(jnp.bfloat16), s_ref[...],
                  preferred_element_type=jnp.float32)
    h = jnp.maximum(acc + b_ref[...], 0.0)
    sn = jnp.dot(h, w_ref[...], preferred_element_type=jnp.float32)
    snext_ref[...] = sn.astype(jnp.bfloat16)
    _stats_accumulate(sn, cs_ref, am_ref)


def _quant_body(s_ref, am_ref, q_ref):
    scale = jnp.maximum(jnp.max(am_ref[...]), 1e-20) / 127.0
    q_ref[...] = jnp.clip(jnp.round(s_ref[...].astype(jnp.float32) / scale),
                          -127.0, 127.0).astype(jnp.int8)


def _dequant_acc(qa_ref, qs_ref, am_ref, cs_ref, b_ref):
    sigma = jnp.maximum(jnp.max(am_ref[...]), 1e-20) / 127.0
    acc = jnp.dot(qa_ref[...], qs_ref[...],
                  preferred_element_type=jnp.int32).astype(jnp.float32)
    return acc * (sigma / 254.0) + 0.5 * cs_ref[...] + b_ref[...]


def _qlayer_body(qa_ref, qs_ref, am_ref, cs_ref, b_ref, w_ref,
                 snext_ref, cs2_ref, am2_ref):
    h = jnp.maximum(_dequant_acc(qa_ref, qs_ref, am_ref, cs_ref, b_ref), 0.0)
    sn = jnp.dot(h, w_ref[...], preferred_element_type=jnp.float32)
    snext_ref[...] = sn.astype(jnp.bfloat16)
    _stats_accumulate(sn, cs2_ref, am2_ref)


def _qfinal_body(qa_ref, qs_ref, am_ref, cs_ref, b_ref, o_ref):
    o_ref[...] = _dequant_acc(qa_ref, qs_ref, am_ref, cs_ref, b_ref)


def _support(h, w):
    fo = w.shape[1]
    return pl.pallas_call(
        _support_body,
        out_shape=jax.ShapeDtypeStruct((N, fo), jnp.bfloat16),
    )(h, w)


def _layer1(adj, s, b, w_next):
    fi = s.shape[1]
    fo = w_next.shape[1]
    return pl.pallas_call(
        _layer1_body,
        grid=(N // BI,),
        in_specs=[
            pl.BlockSpec((BI, N), lambda i: (i, 0)),
            pl.BlockSpec((N, fi), lambda i: (0, 0)),
            pl.BlockSpec((1, fi), lambda i: (0, 0)),
            pl.BlockSpec((fi, fo), lambda i: (0, 0)),
        ],
        out_specs=[
            pl.BlockSpec((BI, fo), lambda i: (i, 0)),
            pl.BlockSpec((1, fo), lambda i: (0, 0)),
            pl.BlockSpec((1, fo), lambda i: (0, 0)),
            pl.BlockSpec((BI, N), lambda i: (i, 0)),
        ],
        out_shape=[
            jax.ShapeDtypeStruct((N, fo), jnp.bfloat16),
            jax.ShapeDtypeStruct((1, fo), jnp.float32),
            jax.ShapeDtypeStruct((1, fo), jnp.float32),
            jax.ShapeDtypeStruct((N, N), jnp.int8),
        ],
    )(adj, s, b.reshape(1, fi), w_next)


def _quant(s, am):
    fo = s.shape[1]
    return pl.pallas_call(
        _quant_body,
        out_shape=jax.ShapeDtypeStruct((N, fo), jnp.int8),
    )(s, am)


def _qlayer(qadj, qs, am, cs, b, w_next):
    fi = qs.shape[1]
    fo = w_next.shape[1]
    return pl.pallas_call(
        _qlayer_body,
        grid=(N // BI,),
        in_specs=[
            pl.BlockSpec((BI, N), lambda i: (i, 0)),
            pl.BlockSpec((N, fi), lambda i: (0, 0)),
            pl.BlockSpec((1, fi), lambda i: (0, 0)),
            pl.BlockSpec((1, fi), lambda i: (0, 0)),
            pl.BlockSpec((1, fi), lambda i: (0, 0)),
            pl.BlockSpec((fi, fo), lambda i: (0, 0)),
        ],
        out_specs=[
            pl.BlockSpec((BI, fo), lambda i: (i, 0)),
            pl.BlockSpec((1, fo), lambda i: (0, 0)),
            pl.BlockSpec((1, fo), lambda i: (0, 0)),
        ],
        out_shape=[
            jax.ShapeDtypeStruct((N, fo), jnp.bfloat16),
            jax.ShapeDtypeStruct((1, fo), jnp.float32),
            jax.ShapeDtypeStruct((1, fo), jnp.float32),
        ],
    )(qadj, qs, am, cs, b.reshape(1, fi), w_next)


def _qfinal(qadj, qs, am, cs, b):
    fo = qs.shape[1]
    return pl.pallas_call(
        _qfinal_body,
        grid=(N // BI,),
        in_specs=[
            pl.BlockSpec((BI, N), lambda i: (i, 0)),
            pl.BlockSpec((N, fo), lambda i: (0, 0)),
            pl.BlockSpec((1, fo), lambda i: (0, 0)),
            pl.BlockSpec((1, fo), lambda i: (0, 0)),
            pl.BlockSpec((1, fo), lambda i: (0, 0)),
        ],
        out_specs=pl.BlockSpec((BI, fo), lambda i: (i, 0)),
        out_shape=jax.ShapeDtypeStruct((N, fo), jnp.float32),
    )(qadj, qs, am, cs, b.reshape(1, fo))


def kernel(x, adj, W1, b1, W2, b2, W3, b3, W4, b4, W5, b5, W6, b6):
    s = _support(x, W1)
    s, cs, am, qadj = _layer1(adj, s, b1, W2)
    for b, w in ((b2, W3), (b3, W4), (b4, W5), (b5, W6)):
        qs = _quant(s, am)
        s, cs, am = _qlayer(qadj, qs, am, cs, b, w)
    qs = _quant(s, am)
    return _qfinal(qadj, qs, am, cs, b6)


# speedup vs baseline: 1.7665x; 1.2143x over previous
import jax
import jax.numpy as jnp
from jax.experimental import pallas as pl

N = 10000
BI = 400


def _support_body(h_ref, w_ref, o_ref):
    o_ref[...] = jnp.dot(h_ref[...], w_ref[...],
                         preferred_element_type=jnp.float32).astype(jnp.bfloat16)


def _stats_accumulate(sn, cs_ref, am_ref):
    cs = jnp.sum(sn, axis=0, keepdims=True)
    am = jnp.max(jnp.abs(sn), axis=0, keepdims=True)
    i = pl.program_id(0)

    @pl.when(i == 0)
    def _():
        cs_ref[...] = cs
        am_ref[...] = am

    @pl.when(i > 0)
    def _():
        cs_ref[...] = cs_ref[...] + cs
        am_ref[...] = jnp.maximum(am_ref[...], am)


def _layer1_body(adj_ref, s_ref, b_ref, w_ref,
                 snext_ref, cs_ref, am_ref, qadj_ref):
    a32 = adj_ref[...]
    qadj_ref[...] = (a32 - 0.5).astype(jnp.float8_e4m3fn)
    acc = jnp.dot(a32.astype(jnp.bfloat16), s_ref[...],
                  preferred_element_type=jnp.float32)
    h = jnp.maximum(acc + b_ref[...], 0.0)
    sn = jnp.dot(h, w_ref[...], preferred_element_type=jnp.float32)
    snext_ref[...] = sn.astype(jnp.bfloat16)
    _stats_accumulate(sn, cs_ref, am_ref)


def _quant_body(s_ref, am_ref, q_ref):
    scale = jnp.maximum(jnp.max(am_ref[...]), 1e-20) / 240.0
    q_ref[...] = (s_ref[...].astype(jnp.float32) / scale
                  ).astype(jnp.float8_e4m3fn)


def _dequant_acc(qa_ref, qs_ref, am_ref, cs_ref, b_ref):
    sigma = jnp.maximum(jnp.max(am_ref[...]), 1e-20) / 240.0
    acc = jnp.dot(qa_ref[...], qs_ref[...],
                  preferred_element_type=jnp.float32)
    return acc * sigma + 0.5 * cs_ref[...] + b_ref[...]


def _qlayer_body(qa_ref, qs_ref, am_ref, cs_ref, b_ref, w_ref,
                 snext_ref, cs2_ref, am2_ref):
    h = jnp.maximum(_dequant_acc(qa_ref, qs_ref, am_ref, cs_ref, b_ref), 0.0)
    sn = jnp.dot(h, w_ref[...], preferred_element_type=jnp.float32)
    snext_ref[...] = sn.astype(jnp.bfloat16)
    _stats_accumulate(sn, cs2_ref, am2_ref)


def _qfinal_body(qa_ref, qs_ref, am_ref, cs_ref, b_ref, o_ref):
    o_ref[...] = _dequant_acc(qa_ref, qs_ref, am_ref, cs_ref, b_ref)


def _support(h, w):
    fo = w.shape[1]
    return pl.pallas_call(
        _support_body,
        out_shape=jax.ShapeDtypeStruct((N, fo), jnp.bfloat16),
    )(h, w)


def _layer1(adj, s, b, w_next):
    fi = s.shape[1]
    fo = w_next.shape[1]
    return pl.pallas_call(
        _layer1_body,
        grid=(N // BI,),
        in_specs=[
            pl.BlockSpec((BI, N), lambda i: (i, 0)),
            pl.BlockSpec((N, fi), lambda i: (0, 0)),
            pl.BlockSpec((1, fi), lambda i: (0, 0)),
            pl.BlockSpec((fi, fo), lambda i: (0, 0)),
        ],
        out_specs=[
            pl.BlockSpec((BI, fo), lambda i: (i, 0)),
            pl.BlockSpec((1, fo), lambda i: (0, 0)),
            pl.BlockSpec((1, fo), lambda i: (0, 0)),
            pl.BlockSpec((BI, N), lambda i: (i, 0)),
        ],
        out_shape=[
            jax.ShapeDtypeStruct((N, fo), jnp.bfloat16),
            jax.ShapeDtypeStruct((1, fo), jnp.float32),
            jax.ShapeDtypeStruct((1, fo), jnp.float32),
            jax.ShapeDtypeStruct((N, N), jnp.float8_e4m3fn),
        ],
    )(adj, s, b.reshape(1, fi), w_next)


def _quant(s, am):
    fo = s.shape[1]
    return pl.pallas_call(
        _quant_body,
        out_shape=jax.ShapeDtypeStruct((N, fo), jnp.float8_e4m3fn),
    )(s, am)


def _qlayer(qadj, qs, am, cs, b, w_next):
    fi = qs.shape[1]
    fo = w_next.shape[1]
    return pl.pallas_call(
        _qlayer_body,
        grid=(N // BI,),
        in_specs=[
            pl.BlockSpec((BI, N), lambda i: (i, 0)),
            pl.BlockSpec((N, fi), lambda i: (0, 0)),
            pl.BlockSpec((1, fi), lambda i: (0, 0)),
            pl.BlockSpec((1, fi), lambda i: (0, 0)),
            pl.BlockSpec((1, fi), lambda i: (0, 0)),
            pl.BlockSpec((fi, fo), lambda i: (0, 0)),
        ],
        out_specs=[
            pl.BlockSpec((BI, fo), lambda i: (i, 0)),
            pl.BlockSpec((1, fo), lambda i: (0, 0)),
            pl.BlockSpec((1, fo), lambda i: (0, 0)),
        ],
        out_shape=[
            jax.ShapeDtypeStruct((N, fo), jnp.bfloat16),
            jax.ShapeDtypeStruct((1, fo), jnp.float32),
            jax.ShapeDtypeStruct((1, fo), jnp.float32),
        ],
    )(qadj, qs, am, cs, b.reshape(1, fi), w_next)


def _qfinal(qadj, qs, am, cs, b):
    fo = qs.shape[1]
    return pl.pallas_call(
        _qfinal_body,
        grid=(N // BI,),
        in_specs=[
            pl.BlockSpec((BI, N), lambda i: (i, 0)),
            pl.BlockSpec((N, fo), lambda i: (0, 0)),
            pl.BlockSpec((1, fo), lambda i: (0, 0)),
            pl.BlockSpec((1, fo), lambda i: (0, 0)),
            pl.BlockSpec((1, fo), lambda i: (0, 0)),
        ],
        out_specs=pl.BlockSpec((BI, fo), lambda i: (i, 0)),
        out_shape=jax.ShapeDtypeStruct((N, fo), jnp.float32),
    )(qadj, qs, am, cs, b.reshape(1, fo))


def kernel(x, adj, W1, b1, W2, b2, W3, b3, W4, b4, W5, b5, W6, b6):
    s = _support(x, W1)
    s, cs, am, qadj = _layer1(adj, s, b1, W2)
    for b, w in ((b2, W3), (b3, W4), (b4, W5), (b5, W6)):
        qs = _quant(s, am)
        s, cs, am = _qlayer(qadj, qs, am, cs, b, w)
    qs = _quant(s, am)
    return _qfinal(qadj, qs, am, cs, b6)


# speedup vs baseline: 1.8005x; 1.0192x over previous
import jax
import jax.numpy as jnp
from jax.experimental import pallas as pl
from jax.experimental.pallas import tpu as pltpu

N = 10000
BI = 400
F8 = jnp.float8_e4m3fn


def _support_body(h_ref, w_ref, o_ref):
    o_ref[...] = jnp.dot(h_ref[...], w_ref[...],
                         preferred_element_type=jnp.float32).astype(jnp.bfloat16)


def _stats_accumulate(sn, cs_ref, am_ref):
    cs = jnp.sum(sn, axis=0, keepdims=True)
    am = jnp.max(jnp.abs(sn), axis=0, keepdims=True)
    i = pl.program_id(0)

    @pl.when(i == 0)
    def _():
        cs_ref[...] = cs
        am_ref[...] = am

    @pl.when(i > 0)
    def _():
        cs_ref[...] = cs_ref[...] + cs
        am_ref[...] = jnp.maximum(am_ref[...], am)


def _next_support(h, w_ref, snext_ref, cs_ref, am_ref):
    sn = jnp.dot(h, w_ref[...], preferred_element_type=jnp.float32)
    snext_ref[...] = sn.astype(jnp.bfloat16)
    _stats_accumulate(sn, cs_ref, am_ref)


def _layer1_body(adj_ref, s_ref, b_ref, w_ref,
                 snext_ref, cs_ref, am_ref, qadj_ref):
    a32 = adj_ref[...]
    qadj_ref[...] = (a32 - 0.5).astype(F8)
    acc = jnp.dot(a32.astype(jnp.bfloat16), s_ref[...],
                  preferred_element_type=jnp.float32)
    h = jnp.maximum(acc + b_ref[...], 0.0)
    _next_support(h, w_ref, snext_ref, cs_ref, am_ref)


def _quant_to_scratch(s_ref, sigma, qs_scr):
    @pl.when(pl.program_id(0) == 0)
    def _():
        qs_scr[...] = (s_ref[...].astype(jnp.float32) / sigma).astype(F8)


def _dequant_acc(qa_ref, qs_scr, sigma, cs_ref, b_ref):
    acc = jnp.dot(qa_ref[...], qs_scr[...],
                  preferred_element_type=jnp.float32)
    return acc * sigma + 0.5 * cs_ref[...] + b_ref[...]


def _qlayer_body(qa_ref, s_ref, am_ref, cs_ref, b_ref, w_ref,
                 snext_ref, cs2_ref, am2_ref, qs_scr):
    sigma = jnp.maximum(jnp.max(am_ref[...]), 1e-20) / 240.0
    _quant_to_scratch(s_ref, sigma, qs_scr)
    h = jnp.maximum(_dequant_acc(qa_ref, qs_scr, sigma, cs_ref, b_ref), 0.0)
    _next_support(h, w_ref, snext_ref, cs2_ref, am2_ref)


def _qfinal_body(qa_ref, s_ref, am_ref, cs_ref, b_ref, o_ref, qs_scr):
    sigma = jnp.maximum(jnp.max(am_ref[...]), 1e-20) / 240.0
    _quant_to_scratch(s_ref, sigma, qs_scr)
    o_ref[...] = _dequant_acc(qa_ref, qs_scr, sigma, cs_ref, b_ref)


def _support(h, w):
    fo = w.shape[1]
    return pl.pallas_call(
        _support_body,
        out_shape=jax.ShapeDtypeStruct((N, fo), jnp.bfloat16),
    )(h, w)


def _layer1(adj, s, b, w_next):
    fi = s.shape[1]
    fo = w_next.shape[1]
    return pl.pallas_call(
        _layer1_body,
        grid=(N // BI,),
        in_specs=[
            pl.BlockSpec((BI, N), lambda i: (i, 0)),
            pl.BlockSpec((N, fi), lambda i: (0, 0)),
            pl.BlockSpec((1, fi), lambda i: (0, 0)),
            pl.BlockSpec((fi, fo), lambda i: (0, 0)),
        ],
        out_specs=[
            pl.BlockSpec((BI, fo), lambda i: (i, 0)),
            pl.BlockSpec((1, fo), lambda i: (0, 0)),
            pl.BlockSpec((1, fo), lambda i: (0, 0)),
            pl.BlockSpec((BI, N), lambda i: (i, 0)),
        ],
        out_shape=[
            jax.ShapeDtypeStruct((N, fo), jnp.bfloat16),
            jax.ShapeDtypeStruct((1, fo), jnp.float32),
            jax.ShapeDtypeStruct((1, fo), jnp.float32),
            jax.ShapeDtypeStruct((N, N), F8),
        ],
    )(adj, s, b.reshape(1, fi), w_next)


def _qlayer(qadj, s, am, cs, b, w_next):
    fi = s.shape[1]
    fo = w_next.shape[1]
    return pl.pallas_call(
        _qlayer_body,
        grid=(N // BI,),
        in_specs=[
            pl.BlockSpec((BI, N), lambda i: (i, 0)),
            pl.BlockSpec((N, fi), lambda i: (0, 0)),
            pl.BlockSpec((1, fi), lambda i: (0, 0)),
            pl.BlockSpec((1, fi), lambda i: (0, 0)),
            pl.BlockSpec((1, fi), lambda i: (0, 0)),
            pl.BlockSpec((fi, fo), lambda i: (0, 0)),
        ],
        out_specs=[
            pl.BlockSpec((BI, fo), lambda i: (i, 0)),
            pl.BlockSpec((1, fo), lambda i: (0, 0)),
            pl.BlockSpec((1, fo), lambda i: (0, 0)),
        ],
        out_shape=[
            jax.ShapeDtypeStruct((N, fo), jnp.bfloat16),
            jax.ShapeDtypeStruct((1, fo), jnp.float32),
            jax.ShapeDtypeStruct((1, fo), jnp.float32),
        ],
        scratch_shapes=[pltpu.VMEM((N, fi), F8)],
    )(qadj, s, am, cs, b.reshape(1, fi), w_next)


def _qfinal(qadj, s, am, cs, b):
    fo = s.shape[1]
    return pl.pallas_call(
        _qfinal_body,
        grid=(N // BI,),
        in_specs=[
            pl.BlockSpec((BI, N), lambda i: (i, 0)),
            pl.BlockSpec((N, fo), lambda i: (0, 0)),
            pl.BlockSpec((1, fo), lambda i: (0, 0)),
            pl.BlockSpec((1, fo), lambda i: (0, 0)),
            pl.BlockSpec((1, fo), lambda i: (0, 0)),
        ],
        out_specs=pl.BlockSpec((BI, fo), lambda i: (i, 0)),
        out_shape=jax.ShapeDtypeStruct((N, fo), jnp.float32),
        scratch_shapes=[pltpu.VMEM((N, fo), F8)],
    )(qadj, s, am, cs, b.reshape(1, fo))


def kernel(x, adj, W1, b1, W2, b2, W3, b3, W4, b4, W5, b5, W6, b6):
    s = _support(x, W1)
    s, cs, am, qadj = _layer1(adj, s, b1, W2)
    for b, w in ((b2, W3), (b3, W4), (b4, W5), (b5, W6)):
        s, cs, am = _qlayer(qadj, s, am, cs, b, w)
    return _qfinal(qadj, s, am, cs, b6)


# speedup vs baseline: 2.0379x; 1.1319x over previous
import jax
import jax.numpy as jnp
from jax.experimental import pallas as pl
from jax.experimental.pallas import tpu as pltpu

N = 10000
BI = 400
BQ = 1000
F8 = jnp.float8_e4m3fn


def _support_body(h_ref, w_ref, o_ref):
    o_ref[...] = jnp.dot(h_ref[...], w_ref[...],
                         preferred_element_type=jnp.float32).astype(jnp.bfloat16)


def _stats_accumulate(sn, cs_ref, am_ref):
    cs = jnp.sum(sn, axis=0, keepdims=True)
    am = jnp.max(jnp.abs(sn), axis=0, keepdims=True)
    i = pl.program_id(0)

    @pl.when(i == 0)
    def _():
        cs_ref[...] = cs
        am_ref[...] = am

    @pl.when(i > 0)
    def _():
        cs_ref[...] = cs_ref[...] + cs
        am_ref[...] = jnp.maximum(am_ref[...], am)


def _next_support(h, w_ref, snext_ref, cs_ref, am_ref):
    sn = jnp.dot(h, w_ref[...], preferred_element_type=jnp.float32)
    snext_ref[...] = sn.astype(jnp.bfloat16)
    _stats_accumulate(sn, cs_ref, am_ref)


def _layer1_body(adj_ref, s_ref, b_ref, w_ref,
                 snext_ref, cs_ref, am_ref, qadj_ref):
    a32 = adj_ref[...]
    qadj_ref[...] = (a32 - 0.5).astype(F8)
    acc = jnp.dot(a32.astype(jnp.bfloat16), s_ref[...],
                  preferred_element_type=jnp.float32)
    h = jnp.maximum(acc + b_ref[...], 0.0)
    _next_support(h, w_ref, snext_ref, cs_ref, am_ref)


def _quant_to_scratch(s_ref, sigma, qs_scr):
    @pl.when(pl.program_id(0) == 0)
    def _():
        qs_scr[...] = (s_ref[...].astype(jnp.float32) / sigma).astype(F8)


def _dequant_acc(qa_ref, qs_scr, sigma, cs_ref, b_ref):
    acc = jnp.dot(qa_ref[...], qs_scr[...],
                  preferred_element_type=jnp.float32)
    return acc * sigma + 0.5 * cs_ref[...] + b_ref[...]


def _qlayer_body(qa_ref, s_ref, am_ref, cs_ref, b_ref, w_ref,
                 snext_ref, cs2_ref, am2_ref, qs_scr):
    sigma = jnp.maximum(jnp.max(am_ref[...]), 1e-20) / 240.0
    _quant_to_scratch(s_ref, sigma, qs_scr)
    h = jnp.maximum(_dequant_acc(qa_ref, qs_scr, sigma, cs_ref, b_ref), 0.0)
    _next_support(h, w_ref, snext_ref, cs2_ref, am2_ref)


def _qfinal_body(qa_ref, s_ref, am_ref, cs_ref, b_ref, o_ref, qs_scr):
    sigma = jnp.maximum(jnp.max(am_ref[...]), 1e-20) / 240.0
    _quant_to_scratch(s_ref, sigma, qs_scr)
    o_ref[...] = _dequant_acc(qa_ref, qs_scr, sigma, cs_ref, b_ref)


def _support(h, w):
    fo = w.shape[1]
    return pl.pallas_call(
        _support_body,
        out_shape=jax.ShapeDtypeStruct((N, fo), jnp.bfloat16),
    )(h, w)


def _layer1(adj, s, b, w_next):
    fi = s.shape[1]
    fo = w_next.shape[1]
    return pl.pallas_call(
        _layer1_body,
        grid=(N // BI,),
        in_specs=[
            pl.BlockSpec((BI, N), lambda i: (i, 0)),
            pl.BlockSpec((N, fi), lambda i: (0, 0)),
            pl.BlockSpec((1, fi), lambda i: (0, 0)),
            pl.BlockSpec((fi, fo), lambda i: (0, 0)),
        ],
        out_specs=[
            pl.BlockSpec((BI, fo), lambda i: (i, 0)),
            pl.BlockSpec((1, fo), lambda i: (0, 0)),
            pl.BlockSpec((1, fo), lambda i: (0, 0)),
            pl.BlockSpec((BI, N), lambda i: (i, 0)),
        ],
        out_shape=[
            jax.ShapeDtypeStruct((N, fo), jnp.bfloat16),
            jax.ShapeDtypeStruct((1, fo), jnp.float32),
            jax.ShapeDtypeStruct((1, fo), jnp.float32),
            jax.ShapeDtypeStruct((N, N), F8),
        ],
    )(adj, s, b.reshape(1, fi), w_next)


def _qlayer(qadj, s, am, cs, b, w_next):
    fi = s.shape[1]
    fo = w_next.shape[1]
    return pl.pallas_call(
        _qlayer_body,
        grid=(N // BQ,),
        in_specs=[
            pl.BlockSpec((BQ, N), lambda i: (i, 0)),
            pl.BlockSpec((N, fi), lambda i: (0, 0)),
            pl.BlockSpec((1, fi), lambda i: (0, 0)),
            pl.BlockSpec((1, fi), lambda i: (0, 0)),
            pl.BlockSpec((1, fi), lambda i: (0, 0)),
            pl.BlockSpec((fi, fo), lambda i: (0, 0)),
        ],
        out_specs=[
            pl.BlockSpec((BQ, fo), lambda i: (i, 0)),
            pl.BlockSpec((1, fo), lambda i: (0, 0)),
            pl.BlockSpec((1, fo), lambda i: (0, 0)),
        ],
        out_shape=[
            jax.ShapeDtypeStruct((N, fo), jnp.bfloat16),
            jax.ShapeDtypeStruct((1, fo), jnp.float32),
            jax.ShapeDtypeStruct((1, fo), jnp.float32),
        ],
        scratch_shapes=[pltpu.VMEM((N, fi), F8)],
    )(qadj, s, am, cs, b.reshape(1, fi), w_next)


def _qfinal(qadj, s, am, cs, b):
    fo = s.shape[1]
    return pl.pallas_call(
        _qfinal_body,
        grid=(N // BQ,),
        in_specs=[
            pl.BlockSpec((BQ, N), lambda i: (i, 0)),
            pl.BlockSpec((N, fo), lambda i: (0, 0)),
            pl.BlockSpec((1, fo), lambda i: (0, 0)),
            pl.BlockSpec((1, fo), lambda i: (0, 0)),
            pl.BlockSpec((1, fo), lambda i: (0, 0)),
        ],
        out_specs=pl.BlockSpec((BQ, fo), lambda i: (i, 0)),
        out_shape=jax.ShapeDtypeStruct((N, fo), jnp.float32),
        scratch_shapes=[pltpu.VMEM((N, fo), F8)],
    )(qadj, s, am, cs, b.reshape(1, fo))


def kernel(x, adj, W1, b1, W2, b2, W3, b3, W4, b4, W5, b5, W6, b6):
    s = _support(x, W1)
    s, cs, am, qadj = _layer1(adj, s, b1, W2)
    for b, w in ((b2, W3), (b3, W4), (b4, W5), (b5, W6)):
        s, cs, am = _qlayer(qadj, s, am, cs, b, w)
    return _qfinal(qadj, s, am, cs, b6)
